# baseline (speedup 1.0000x reference)
import jax
import jax.numpy as jnp
from jax.experimental import pallas as pl
from jax.experimental.pallas import tpu as pltpu

_B, _C, _H, _W = 8, 19, 512, 512
_HB = 128


def _body(x_ref, t_ref, out_ref):
    x = x_ref[0]
    t = t_ref[0]
    cls = jax.lax.broadcasted_iota(jnp.int32, x.shape, 0)
    y = (cls == t[None]).astype(jnp.float32)
    loss = jnp.maximum(x, 0.0) - x * y + jnp.log1p(jnp.exp(-jnp.abs(x)))
    s = jnp.sum(loss)

    @pl.when((pl.program_id(0) == 0) & (pl.program_id(1) == 0))
    def _():
        out_ref[0, 0] = 0.0

    out_ref[0, 0] += s


def kernel(input, target, epoch):
    del epoch
    n = input.size
    grid = (_B, _H // _HB)
    out = pl.pallas_call(
        _body,
        grid=grid,
        in_specs=[
            pl.BlockSpec((1, _C, _HB, _W), lambda b, h: (b, 0, h, 0)),
            pl.BlockSpec((1, _HB, _W), lambda b, h: (b, h, 0)),
        ],
        out_specs=pl.BlockSpec(memory_space=pltpu.SMEM),
        out_shape=jax.ShapeDtypeStruct((1, 1), jnp.float32),
    )(input, target)
    return out[0, 0] / n
